# baseline (speedup 1.0000x reference)
import functools

import jax
import jax.numpy as jnp
from jax import lax
from jax.experimental import pallas as pl
from jax.experimental.pallas import tpu as pltpu
from jax.experimental.pallas import tpu_sc as plsc

_B = 16384
_D = 64
_L = 16

_info = plsc.get_sparse_core_info()
_NC, _NS = _info.num_cores, _info.num_subcores
_NW = _NC * _NS
_BPW = _B // _NW
_IDX_CHUNK = 128
_NCHUNK = _BPW // _IDX_CHUNK
_GROUPS = _BPW // _L


def _gather_body(uidx_hbm, iidx_hbm, ubias_hbm, ibias_hbm, out_hbm,
                 uidx_v, iidx_v, ub_v, ib_v, out_v, isem, sem):
    wid = lax.axis_index("s") * _NC + lax.axis_index("c")
    base = wid * _BPW

    cu = pltpu.async_copy(uidx_hbm.at[pl.ds(base, _BPW)], uidx_v, isem)
    ci = pltpu.async_copy(iidx_hbm.at[pl.ds(base, _BPW)], iidx_v, isem)

    copies = []
    cu.wait()
    for j in range(_NCHUNK):
        s = pl.ds(j * _IDX_CHUNK, _IDX_CHUNK)
        copies.append(pltpu.async_copy(
            ubias_hbm.at[0].at[uidx_v.at[s]], ub_v.at[s], sem))
    ci.wait()
    for j in range(_NCHUNK):
        s = pl.ds(j * _IDX_CHUNK, _IDX_CHUNK)
        copies.append(pltpu.async_copy(
            ibias_hbm.at[0].at[iidx_v.at[s]], ib_v.at[s], sem))
    for c in copies:
        c.wait()

    def group(g, _):
        row0 = g * _L
        out_v[pl.ds(row0, _L)] = (ub_v[pl.ds(row0, _L)]
                                  + ib_v[pl.ds(row0, _L)])
        return _

    lax.fori_loop(0, _GROUPS, group, None, unroll=True)
    pltpu.sync_copy(out_v, out_hbm.at[pl.ds(base, _BPW)])


def _dot_body(u_ref, i_ref, o_ref):
    o_ref[...] = jnp.sum(u_ref[...] * i_ref[...], axis=0)


def _add_body(g_ref, a_ref, b_ref, o_ref):
    o_ref[...] = a_ref[...] + b_ref[...] + g_ref[0]


_DOT_BLK = 1024


@jax.jit
def _pooler(u_emb, i_emb, u_idx, i_idx, ubias, ibias, gb):
    mesh = plsc.VectorSubcoreMesh(core_axis_name="c", subcore_axis_name="s")
    bias_sum = functools.partial(
        pl.kernel, mesh=mesh,
        out_type=jax.ShapeDtypeStruct((_B,), jnp.float32),
        scratch_types=[
            pltpu.VMEM((_BPW,), jnp.int32),
            pltpu.VMEM((_BPW,), jnp.int32),
            pltpu.VMEM((_BPW,), jnp.float32),
            pltpu.VMEM((_BPW,), jnp.float32),
            pltpu.VMEM((_BPW,), jnp.float32),
            pltpu.SemaphoreType.DMA,
            pltpu.SemaphoreType.DMA,
        ],
    )(_gather_body)(u_idx, i_idx, ubias, ibias)

    dot = pl.pallas_call(
        _dot_body,
        grid=(_B // _DOT_BLK,),
        in_specs=[
            pl.BlockSpec((_D, _DOT_BLK), lambda j: (0, j)),
            pl.BlockSpec((_D, _DOT_BLK), lambda j: (0, j)),
        ],
        out_specs=pl.BlockSpec((_DOT_BLK,), lambda j: (j,)),
        out_shape=jax.ShapeDtypeStruct((_B,), jnp.float32),
    )(u_emb, i_emb)

    return pl.pallas_call(
        _add_body,
        in_specs=[
            pl.BlockSpec(memory_space=pltpu.SMEM),
            pl.BlockSpec((_B,), lambda: (0,)),
            pl.BlockSpec((_B,), lambda: (0,)),
        ],
        out_shape=jax.ShapeDtypeStruct((_B,), jnp.float32),
    )(gb, dot, bias_sum)


def kernel(u_emb, i_emb, u_idx, i_idx, user_bias, item_bias, global_bias):
    return _pooler(
        u_emb.T, i_emb.T,
        u_idx.astype(jnp.int32), i_idx.astype(jnp.int32),
        user_bias.T, item_bias.T,
        global_bias.astype(jnp.float32))


# speedup vs baseline: 1.1615x; 1.1615x over previous
import functools

import jax
import jax.numpy as jnp
from jax import lax
from jax.experimental import pallas as pl
from jax.experimental.pallas import tpu as pltpu
from jax.experimental.pallas import tpu_sc as plsc

_B = 16384
_D = 64
_L = 16

_info = plsc.get_sparse_core_info()
_NC, _NS = _info.num_cores, _info.num_subcores
_NW = _NC * _NS
_BPW = _B // _NW
_IDX_CHUNK = 128
_NCHUNK = _BPW // _IDX_CHUNK
_GROUPS = _BPW // _L


def _gather_body(uidx_hbm, iidx_hbm, ubias_hbm, ibias_hbm, out_hbm,
                 uidx_v, iidx_v, ub_v, ib_v, out_v, isem, sem):
    wid = lax.axis_index("s") * _NC + lax.axis_index("c")
    base = wid * _BPW

    cu = pltpu.async_copy(uidx_hbm.at[pl.ds(base, _BPW)], uidx_v, isem)
    ci = pltpu.async_copy(iidx_hbm.at[pl.ds(base, _BPW)], iidx_v, isem)

    copies = []
    cu.wait()
    for j in range(_NCHUNK):
        s = pl.ds(j * _IDX_CHUNK, _IDX_CHUNK)
        copies.append(pltpu.async_copy(
            ubias_hbm.at[0].at[uidx_v.at[s]], ub_v.at[s], sem))
    ci.wait()
    for j in range(_NCHUNK):
        s = pl.ds(j * _IDX_CHUNK, _IDX_CHUNK)
        copies.append(pltpu.async_copy(
            ibias_hbm.at[0].at[iidx_v.at[s]], ib_v.at[s], sem))
    for c in copies:
        c.wait()

    def group(g, _):
        row0 = g * _L
        out_v[pl.ds(row0, _L)] = (ub_v[pl.ds(row0, _L)]
                                  + ib_v[pl.ds(row0, _L)])
        return _

    lax.fori_loop(0, _GROUPS, group, None, unroll=True)
    pltpu.sync_copy(out_v, out_hbm.at[pl.ds(base, _BPW)])


def _dot_body(u_ref, i_ref, o_ref):
    o_ref[...] = jnp.sum(u_ref[...] * i_ref[...], axis=0)


def _add_body(g_ref, a_ref, b_ref, o_ref):
    o_ref[...] = a_ref[...] + b_ref[...] + g_ref[0]


_DOT_BLK = 4096


@jax.jit
def _pooler(u_emb, i_emb, u_idx, i_idx, ubias, ibias, gb):
    mesh = plsc.VectorSubcoreMesh(core_axis_name="c", subcore_axis_name="s")
    bias_sum = functools.partial(
        pl.kernel, mesh=mesh,
        out_type=jax.ShapeDtypeStruct((_B,), jnp.float32),
        scratch_types=[
            pltpu.VMEM((_BPW,), jnp.int32),
            pltpu.VMEM((_BPW,), jnp.int32),
            pltpu.VMEM((_BPW,), jnp.float32),
            pltpu.VMEM((_BPW,), jnp.float32),
            pltpu.VMEM((_BPW,), jnp.float32),
            pltpu.SemaphoreType.DMA,
            pltpu.SemaphoreType.DMA,
        ],
    )(_gather_body)(u_idx, i_idx, ubias, ibias)

    dot = pl.pallas_call(
        _dot_body,
        grid=(_B // _DOT_BLK,),
        in_specs=[
            pl.BlockSpec((_D, _DOT_BLK), lambda j: (0, j)),
            pl.BlockSpec((_D, _DOT_BLK), lambda j: (0, j)),
        ],
        out_specs=pl.BlockSpec((_DOT_BLK,), lambda j: (j,)),
        out_shape=jax.ShapeDtypeStruct((_B,), jnp.float32),
    )(u_emb, i_emb)

    return pl.pallas_call(
        _add_body,
        in_specs=[
            pl.BlockSpec(memory_space=pltpu.SMEM),
            pl.BlockSpec((_B,), lambda: (0,)),
            pl.BlockSpec((_B,), lambda: (0,)),
        ],
        out_shape=jax.ShapeDtypeStruct((_B,), jnp.float32),
    )(gb, dot, bias_sum)


def kernel(u_emb, i_emb, u_idx, i_idx, user_bias, item_bias, global_bias):
    return _pooler(
        u_emb.T, i_emb.T,
        u_idx.astype(jnp.int32), i_idx.astype(jnp.int32),
        user_bias.T, item_bias.T,
        global_bias.astype(jnp.float32))
